# baseline (speedup 1.0000x reference)
import functools

import jax
import jax.numpy as jnp
from jax import lax
from jax.experimental import pallas as pl
from jax.experimental.pallas import tpu as pltpu
from jax.experimental.pallas import tpu_sc as plsc

_PATCH = 16
_LANES = 16
_NC = 2
_NS = 16
_NW = _NC * _NS
_CHUNK = 128


def _src_rows(C, H, W):
    p = _PATCH
    h, w = H // p, W // p
    perm = jax.random.permutation(jax.random.key(42), h * w)
    sh = (perm // w).reshape(h, w)
    sw = (perm % w).reshape(h, w)
    c_b = jnp.arange(C, dtype=jnp.int32)[:, None, None, None]
    r_b = jnp.arange(p, dtype=jnp.int32)[None, None, :, None]
    src = ((c_b * h + sh[None, :, None, :]) * p + r_b) * w + sw[None, :, None, :]
    return src.reshape(-1).astype(jnp.int32)


def _make_gather(num_rows):
    rows_per_w = num_rows // _NW
    n_chunks = rows_per_w // _CHUNK
    mesh = plsc.VectorSubcoreMesh(core_axis_name="c", subcore_axis_name="s")

    @functools.partial(
        pl.kernel,
        mesh=mesh,
        out_type=jax.ShapeDtypeStruct((num_rows, _LANES), jnp.float32),
        scratch_types=[
            pltpu.VMEM((n_chunks, _CHUNK), jnp.int32),
            pltpu.VMEM((_CHUNK, _LANES), jnp.float32),
            pltpu.SemaphoreType.DMA,
        ],
        compiler_params=pltpu.CompilerParams(use_tc_tiling_on_sc=False),
    )
    def gather(table_hbm, idx_hbm, out_hbm, idx_v, rows_v, sem):
        wid = lax.axis_index("s") * _NC + lax.axis_index("c")
        base = wid * rows_per_w
        pltpu.sync_copy(idx_hbm.at[wid], idx_v)

        def body(j, carry):
            pltpu.async_copy(table_hbm.at[idx_v.at[j]], rows_v, sem).wait()
            pltpu.sync_copy(rows_v, out_hbm.at[pl.ds(base + j * _CHUNK, _CHUNK)])
            return carry

        lax.fori_loop(0, n_chunks, body, 0)

    return gather


def kernel(image):
    C, H, W = image.shape
    num_rows = C * H * (W // _PATCH)
    table = image.reshape(num_rows, _LANES)
    idx = _src_rows(C, H, W).reshape(_NW, num_rows // (_NW * _CHUNK), _CHUNK)
    out = _make_gather(num_rows)(table, idx)
    return out.reshape(C, H, W)


# speedup vs baseline: 1.8102x; 1.8102x over previous
import functools

import jax
import jax.numpy as jnp
from jax import lax
from jax.experimental import pallas as pl
from jax.experimental.pallas import tpu as pltpu
from jax.experimental.pallas import tpu_sc as plsc

_PATCH = 16
_LANES = 16
_NC = 2
_NS = 16
_NW = _NC * _NS
_CHUNK = 128


def _src_rows(C, H, W):
    p = _PATCH
    h, w = H // p, W // p
    perm = jax.random.permutation(jax.random.key(42), h * w)
    sh = (perm // w).reshape(h, w)
    sw = (perm % w).reshape(h, w)
    c_b = jnp.arange(C, dtype=jnp.int32)[:, None, None, None]
    r_b = jnp.arange(p, dtype=jnp.int32)[None, None, :, None]
    src = ((c_b * h + sh[None, :, None, :]) * p + r_b) * w + sw[None, :, None, :]
    return src.reshape(-1).astype(jnp.int32)


_NBUF = 8


def _make_gather(num_rows):
    rows_per_w = num_rows // _NW
    n_chunks = rows_per_w // _CHUNK
    n_groups = n_chunks // _NBUF
    mesh = plsc.VectorSubcoreMesh(core_axis_name="c", subcore_axis_name="s")

    @functools.partial(
        pl.kernel,
        mesh=mesh,
        out_type=jax.ShapeDtypeStruct((num_rows, _LANES), jnp.float32),
        scratch_types=[
            pltpu.VMEM((n_chunks, _CHUNK), jnp.int32),
            pltpu.VMEM((_NBUF, _CHUNK, _LANES), jnp.float32),
            pltpu.SemaphoreType.DMA,
            pltpu.SemaphoreType.DMA,
        ],
        compiler_params=pltpu.CompilerParams(use_tc_tiling_on_sc=False),
    )
    def gather(table_hbm, idx_hbm, out_hbm, idx_v, rows_v, sem_g, sem_s):
        wid = lax.axis_index("s") * _NC + lax.axis_index("c")
        base = wid * rows_per_w
        pltpu.sync_copy(idx_hbm.at[wid], idx_v)

        def out_slice(j):
            return out_hbm.at[pl.ds(base + j * _CHUNK, _CHUNK)]

        def body(g, carry):
            for b in range(_NBUF):
                @pl.when(g > 0)
                def _():
                    pltpu.make_async_copy(rows_v.at[b], out_slice(0), sem_s).wait()

                j = g * _NBUF + b
                pltpu.async_copy(table_hbm.at[idx_v.at[j]], rows_v.at[b], sem_g)
            for b in range(_NBUF):
                j = g * _NBUF + b
                pltpu.make_async_copy(table_hbm.at[idx_v.at[j]], rows_v.at[b],
                                      sem_g).wait()
                pltpu.async_copy(rows_v.at[b], out_slice(j), sem_s)
            return carry

        lax.fori_loop(0, n_groups, body, 0)
        for b in range(_NBUF):
            pltpu.make_async_copy(rows_v.at[b], out_slice(0), sem_s).wait()

    return gather


def kernel(image):
    C, H, W = image.shape
    num_rows = C * H * (W // _PATCH)
    table = image.reshape(num_rows, _LANES)
    idx = _src_rows(C, H, W).reshape(_NW, num_rows // (_NW * _CHUNK), _CHUNK)
    out = _make_gather(num_rows)(table, idx)
    return out.reshape(C, H, W)


# speedup vs baseline: 2.0494x; 1.1321x over previous
import functools

import jax
import jax.numpy as jnp
from jax import lax
from jax.experimental import pallas as pl
from jax.experimental.pallas import tpu as pltpu
from jax.experimental.pallas import tpu_sc as plsc

_PATCH = 16
_LANES = 16
_NC = 2
_NS = 16
_NW = _NC * _NS
_CHUNK = 128


def _src_rows(C, H, W):
    p = _PATCH
    h, w = H // p, W // p
    perm = jax.random.permutation(jax.random.key(42), h * w)
    sh = (perm // w).reshape(h, w)
    sw = (perm % w).reshape(h, w)
    c_b = jnp.arange(C, dtype=jnp.int32)[:, None, None, None]
    r_b = jnp.arange(p, dtype=jnp.int32)[None, None, :, None]
    src = ((c_b * h + sh[None, :, None, :]) * p + r_b) * w + sw[None, :, None, :]
    return src.reshape(-1).astype(jnp.int32)


_NBUF = 16


def _make_gather(num_rows):
    rows_per_w = num_rows // _NW
    n_chunks = rows_per_w // _CHUNK
    n_groups = n_chunks // _NBUF
    mesh = plsc.VectorSubcoreMesh(core_axis_name="c", subcore_axis_name="s")

    @functools.partial(
        pl.kernel,
        mesh=mesh,
        out_type=jax.ShapeDtypeStruct((num_rows, _LANES), jnp.float32),
        scratch_types=[
            pltpu.VMEM((n_chunks, _CHUNK), jnp.int32),
            pltpu.VMEM((_NBUF, _CHUNK, _LANES), jnp.float32),
            pltpu.SemaphoreType.DMA,
            pltpu.SemaphoreType.DMA,
        ],
        compiler_params=pltpu.CompilerParams(use_tc_tiling_on_sc=False),
    )
    def gather(table_hbm, idx_hbm, out_hbm, idx_v, rows_v, sem_g, sem_s):
        wid = lax.axis_index("s") * _NC + lax.axis_index("c")
        base = wid * rows_per_w
        pltpu.sync_copy(idx_hbm.at[wid], idx_v)

        def out_slice(j):
            return out_hbm.at[pl.ds(base + j * _CHUNK, _CHUNK)]

        def body(g, carry):
            for b in range(_NBUF):
                @pl.when(g > 0)
                def _():
                    pltpu.make_async_copy(rows_v.at[b], out_slice(0), sem_s).wait()

                j = g * _NBUF + b
                pltpu.async_copy(table_hbm.at[idx_v.at[j]], rows_v.at[b], sem_g)
            for b in range(_NBUF):
                j = g * _NBUF + b
                pltpu.make_async_copy(table_hbm.at[idx_v.at[j]], rows_v.at[b],
                                      sem_g).wait()
                pltpu.async_copy(rows_v.at[b], out_slice(j), sem_s)
            return carry

        lax.fori_loop(0, n_groups, body, 0)
        for b in range(_NBUF):
            pltpu.make_async_copy(rows_v.at[b], out_slice(0), sem_s).wait()

    return gather


def kernel(image):
    C, H, W = image.shape
    num_rows = C * H * (W // _PATCH)
    table = image.reshape(num_rows, _LANES)
    idx = _src_rows(C, H, W).reshape(_NW, num_rows // (_NW * _CHUNK), _CHUNK)
    out = _make_gather(num_rows)(table, idx)
    return out.reshape(C, H, W)
